# Initial kernel scaffold; baseline (speedup 1.0000x reference)
import jax
import jax.numpy as jnp
from jax.experimental import pallas as pl


def kernel(x, edge_index, edge_attr1, edge_attr2, batch, params):
    raise NotImplementedError("write your pallas kernel here")


# speedup vs baseline: 1.7660x; 1.7660x over previous
import functools

import jax
import jax.numpy as jnp
from jax.experimental import pallas as pl

_N = 2048
_E = 8192
_G = 64
_BLK = 512
_KC = 16


def _layer_body(x_ref, src_ref, dst_ref, ea_ref, w0_ref, b0_ref, w1_ref,
                b1_ref, w2r_ref, b2r_ref, root_ref, bias_ref, out_ref,
                *, din, dout):
    x = x_ref[...]
    out_ref[...] = x @ root_ref[...] + bias_ref[...]

    def body(b, _):
        sl = pl.ds(b * _BLK, _BLK)
        sb = src_ref[sl]
        db = dst_ref[sl]
        iota_bn = jax.lax.broadcasted_iota(jnp.int32, (_BLK, _N), 1)
        oh_s = (jnp.reshape(sb, (_BLK, 1)) == iota_bn).astype(jnp.float32)
        xs = oh_s @ x
        ea = ea_ref[sl, :]
        h = jax.nn.relu(ea @ w0_ref[...] + b0_ref[...])
        h = jax.nn.relu(h @ w1_ref[...] + b1_ref[...])
        msg = xs @ b2r_ref[...]
        if din == 4:
            t = h @ w2r_ref[...]
            for i in range(din):
                msg = msg + xs[:, i:i + 1] * t[:, i * dout:(i + 1) * dout]
        else:
            for k0 in range(0, 64, _KC):
                hc = h[:, k0:k0 + _KC]
                hrep = jnp.broadcast_to(hc[:, :, None],
                                        (_BLK, _KC, din)).reshape(_BLK, _KC * din)
                xst = jnp.broadcast_to(xs[:, None, :],
                                       (_BLK, _KC, din)).reshape(_BLK, _KC * din)
                msg = msg + (hrep * xst) @ w2r_ref[pl.ds(k0 * din, _KC * din), :]
        iota_nb = jax.lax.broadcasted_iota(jnp.int32, (_N, _BLK), 0)
        oh_dt = (jnp.reshape(db, (1, _BLK)) == iota_nb).astype(jnp.float32)
        out_ref[...] += oh_dt @ msg
        return 0

    jax.lax.fori_loop(0, _E // _BLK, body, 0)
    out_ref[...] = jax.nn.relu(out_ref[...])


def _nnconv(x, src, dst, ea, p):
    din, dout = p['root'].shape
    w2r = p['w2'] if din == 4 else p['w2'].reshape(64 * din, dout)
    f = pl.pallas_call(
        functools.partial(_layer_body, din=din, dout=dout),
        out_shape=jax.ShapeDtypeStruct((_N, dout), jnp.float32),
    )
    return f(x, src, dst, ea,
             p['w0'], p['b0'].reshape(1, 64),
             p['w1'], p['b1'].reshape(1, 64),
             w2r, p['b2'].reshape(din, dout),
             p['root'], p['bias'].reshape(1, dout))


def _head_body(x1_ref, x2_ref, iin_ref, iout_ref, l1w_ref, l1b_ref, l2w_ref,
               l2b_ref, l3w_ref, l3b_ref, ow_ref, ob_ref, out_ref):
    wl = (l1w_ref[...] @ l2w_ref[...]) @ l3w_ref[...]
    bl = (l1b_ref[...] @ l2w_ref[...] + l2b_ref[...]) @ l3w_ref[...] + l3b_ref[...]
    iota_gn = jax.lax.broadcasted_iota(jnp.int32, (_G, _N), 1)
    oh_in = (jnp.reshape(iin_ref[...], (_G, 1)) == iota_gn).astype(jnp.float32)
    oh_out = (jnp.reshape(iout_ref[...], (_G, 1)) == iota_gn).astype(jnp.float32)
    x1 = x1_ref[...]
    x2 = x2_ref[...]
    a_in = jnp.concatenate([oh_in @ x1, oh_in @ x2], axis=1) @ wl + bl
    a_out = jnp.concatenate([oh_out @ x1, oh_out @ x2], axis=1) @ wl + bl
    cat = jnp.concatenate([a_in, a_out], axis=1)
    y = jnp.sum(cat * ow_ref[...], axis=1, keepdims=True) + ob_ref[...]
    out_ref[...] = y


def kernel(x, edge_index, edge_attr1, edge_attr2, batch, params):
    src = edge_index[0]
    dst = edge_index[1]
    counts = jnp.bincount(batch, length=_G)
    starts = (jnp.cumsum(counts) - counts).astype(jnp.int32)
    input_ind = starts
    output_ind = starts + 1

    x1 = x
    for name in ('c1a', 'c1b', 'c1c', 'c1d'):
        x1 = _nnconv(x1, src, dst, edge_attr1, params[name])
    x2 = x
    for name in ('c2a', 'c2b', 'c2c', 'c2d'):
        x2 = _nnconv(x2, src, dst, edge_attr2, params[name])

    head = pl.pallas_call(
        _head_body,
        out_shape=jax.ShapeDtypeStruct((_G, 1), jnp.float32),
    )
    return head(x1, x2, input_ind, output_ind,
                params['lin1_w'], params['lin1_b'].reshape(1, 128),
                params['lin2_w'], params['lin2_b'].reshape(1, 64),
                params['lin3_w'], params['lin3_b'].reshape(1, 64),
                params['out_w'].reshape(1, 128), params['out_b'].reshape(1, 1))


# speedup vs baseline: 1.8981x; 1.0748x over previous
import functools

import jax
import jax.numpy as jnp
from jax import lax
from jax.experimental import pallas as pl
from jax.experimental.pallas import tpu as pltpu
from jax.experimental.pallas import tpu_sc as plsc

_N = 2048
_E = 8192
_G = 64
_W = 128
_BLK = 512
_KC = 16

_NW = 32
_EPW = _E // _NW
_EPT = _E // 16
_NH = _N // 2
_NPT = _NH // 16


def _sc_mesh():
    return plsc.VectorSubcoreMesh(core_axis_name="c", subcore_axis_name="s")


def _make_gather():
    @functools.partial(
        pl.kernel, mesh=_sc_mesh(),
        out_type=jax.ShapeDtypeStruct((_E, _W), jnp.float32),
        scratch_types=[
            pltpu.VMEM((2, 128), jnp.int32),
            pltpu.VMEM((_EPW, _W), jnp.float32),
            pltpu.SemaphoreType.DMA,
        ],
    )
    def g(x_hbm, src_hbm, out_hbm, idx_v, rows_v, sem):
        wid = lax.axis_index("s") * 2 + lax.axis_index("c")
        pltpu.sync_copy(src_hbm.at[pl.ds(wid * 2, 2)], idx_v)
        for j in range(2):
            pltpu.async_copy(x_hbm.at[idx_v.at[j]],
                             rows_v.at[pl.ds(j * 128, 128)], sem).wait()
        pltpu.sync_copy(rows_v, out_hbm.at[pl.ds(wid * _EPW, _EPW)])

    return g


def _make_scatter():
    @functools.partial(
        pl.kernel, mesh=_sc_mesh(),
        out_type=jax.ShapeDtypeStruct((_N, _W), jnp.float32),
        scratch_types=[
            pltpu.VMEM_SHARED((_NH + 8, _W), jnp.float32),
            pltpu.VMEM((_EPT, _W), jnp.float32),
            pltpu.VMEM((4, 128), jnp.int32),
            pltpu.SemaphoreType.DMA,
        ],
    )
    def s(msg_hbm, dst_hbm, root_hbm, out_hbm, acc, buf, idx_v, sem):
        c = lax.axis_index("c")
        sid = lax.axis_index("s")
        row0 = c * _NH + sid * _NPT
        pltpu.sync_copy(root_hbm.at[pl.ds(row0, _NPT)],
                        acc.at[pl.ds(sid * _NPT, _NPT)])
        plsc.subcore_barrier()
        pltpu.sync_copy(dst_hbm.at[c, pl.ds(sid * 4, 4)], idx_v)
        pltpu.sync_copy(msg_hbm.at[pl.ds(sid * _EPT, _EPT)], buf)
        for j in range(4):
            pltpu.sync_copy(buf.at[pl.ds(j * 128, 128)], acc.at[idx_v.at[j]],
                            add=True)
        plsc.subcore_barrier()
        pltpu.sync_copy(acc.at[pl.ds(sid * _NPT, _NPT)],
                        out_hbm.at[pl.ds(row0, _NPT)])

    return s


def _msg_body(xs_ref, x_ref, ea_ref, dst_ref, w0_ref, b0_ref, w1_ref, b1_ref,
              w2r_ref, b2r_ref, root_ref, bias_ref, out_ref,
              *, din, dout, first):
    x = x_ref[...]
    if not first:
        x = jax.nn.relu(x)[:, :din]
    out_ref[...] = x @ root_ref[...] + bias_ref[...]

    def body(b, _):
        sl = pl.ds(b * _BLK, _BLK)
        xs = xs_ref[sl, :]
        if not first:
            xs = jax.nn.relu(xs)
        xs = xs[:, :din]
        ea = ea_ref[sl, :]
        h = jax.nn.relu(ea @ w0_ref[...] + b0_ref[...])
        h = jax.nn.relu(h @ w1_ref[...] + b1_ref[...])
        msg = xs @ b2r_ref[...]
        if din == 4:
            t = h @ w2r_ref[...]
            for i in range(din):
                msg = msg + xs[:, i:i + 1] * t[:, i * dout:(i + 1) * dout]
        else:
            for k0 in range(0, 64, _KC):
                hc = h[:, k0:k0 + _KC]
                hrep = jnp.broadcast_to(hc[:, :, None],
                                        (_BLK, _KC, din)).reshape(_BLK, _KC * din)
                xst = jnp.broadcast_to(xs[:, None, :],
                                       (_BLK, _KC, din)).reshape(_BLK, _KC * din)
                msg = msg + (hrep * xst) @ w2r_ref[pl.ds(k0 * din, _KC * din), :]
        if dout < _W:
            msg = jnp.concatenate(
                [msg, jnp.zeros((_BLK, _W - dout), jnp.float32)], axis=1)
        db = dst_ref[sl]
        iota_nb = jax.lax.broadcasted_iota(jnp.int32, (_N, _BLK), 0)
        oh_dt = (jnp.reshape(db, (1, _BLK)) == iota_nb).astype(jnp.float32)
        out_ref[...] += oh_dt @ msg
        return 0

    jax.lax.fori_loop(0, _E // _BLK, body, 0)


def _nnconv(x_raw, xs_raw, dst, ea, p, scatter, first):
    din, dout = p['root'].shape
    w2r = p['w2'] if din == 4 else p['w2'].reshape(64 * din, dout)
    root_p = jnp.pad(p['root'], ((0, 0), (0, _W - dout)))
    bias_p = jnp.pad(p['bias'], (0, _W - dout)).reshape(1, _W)
    f = pl.pallas_call(
        functools.partial(_msg_body, din=din, dout=dout, first=first),
        out_shape=jax.ShapeDtypeStruct((_N, _W), jnp.float32),
    )
    return f(xs_raw, x_raw, ea, dst,
             p['w0'], p['b0'].reshape(1, 64),
             p['w1'], p['b1'].reshape(1, 64),
             w2r, p['b2'].reshape(din, dout),
             root_p, bias_p)


def _head_body(x1_ref, x2_ref, iin_ref, iout_ref, l1w_ref, l1b_ref, l2w_ref,
               l2b_ref, l3w_ref, l3b_ref, ow_ref, ob_ref, out_ref):
    wl = (l1w_ref[...] @ l2w_ref[...]) @ l3w_ref[...]
    bl = (l1b_ref[...] @ l2w_ref[...] + l2b_ref[...]) @ l3w_ref[...] + l3b_ref[...]
    iota_gn = jax.lax.broadcasted_iota(jnp.int32, (_G, _N), 1)
    oh_in = (jnp.reshape(iin_ref[...], (_G, 1)) == iota_gn).astype(jnp.float32)
    oh_out = (jnp.reshape(iout_ref[...], (_G, 1)) == iota_gn).astype(jnp.float32)
    x1 = jax.nn.relu(x1_ref[...])[:, :64]
    x2 = jax.nn.relu(x2_ref[...])[:, :64]
    a_in = jnp.concatenate([oh_in @ x1, oh_in @ x2], axis=1) @ wl + bl
    a_out = jnp.concatenate([oh_out @ x1, oh_out @ x2], axis=1) @ wl + bl
    cat = jnp.concatenate([a_in, a_out], axis=1)
    out_ref[...] = jnp.sum(cat * ow_ref[...], axis=1, keepdims=True) + ob_ref[...]


def kernel(x, edge_index, edge_attr1, edge_attr2, batch, params):
    src2d = edge_index[0].reshape(_E // 128, 128)
    dst = edge_index[1]
    counts = jnp.bincount(batch, length=_G)
    starts = (jnp.cumsum(counts) - counts).astype(jnp.int32)

    x0p = jnp.pad(x, ((0, 0), (0, _W - x.shape[1])))
    gather = _make_gather()

    def chain(ea, names):
        xc = x
        xc_pad = x0p
        first = True
        for name in names:
            xs = gather(xc_pad, src2d)
            xc = _nnconv(xc, xs, dst, ea, params[name], None, first)
            xc_pad = xc
            first = False
        return xc

    x1 = chain(edge_attr1, ('c1a', 'c1b', 'c1c', 'c1d'))
    x2 = chain(edge_attr2, ('c2a', 'c2b', 'c2c', 'c2d'))

    head = pl.pallas_call(
        _head_body,
        out_shape=jax.ShapeDtypeStruct((_G, 1), jnp.float32),
    )
    return head(x1, x2, starts, starts + 1,
                params['lin1_w'], params['lin1_b'].reshape(1, 128),
                params['lin2_w'], params['lin2_b'].reshape(1, 64),
                params['lin3_w'], params['lin3_b'].reshape(1, 64),
                params['out_w'].reshape(1, 128), params['out_b'].reshape(1, 1))


# speedup vs baseline: 2.0710x; 1.0911x over previous
import functools

import jax
import jax.numpy as jnp
from jax import lax
from jax.experimental import pallas as pl
from jax.experimental.pallas import tpu as pltpu
from jax.experimental.pallas import tpu_sc as plsc

_N = 2048
_E = 8192
_G = 64
_W = 128
_BLK = 512
_KC = 16

_NW = 32
_EPW = _E // _NW
_EPT = _E // 16
_NH = _N // 2
_NPT = _NH // 16


def _sc_mesh():
    return plsc.VectorSubcoreMesh(core_axis_name="c", subcore_axis_name="s")


def _make_gather():
    @functools.partial(
        pl.kernel, mesh=_sc_mesh(),
        out_type=jax.ShapeDtypeStruct((_E, _W), jnp.float32),
        scratch_types=[
            pltpu.VMEM((2, 128), jnp.int32),
            pltpu.VMEM((_EPW, _W), jnp.float32),
            pltpu.SemaphoreType.DMA,
        ],
    )
    def g(x_hbm, src_hbm, out_hbm, idx_v, rows_v, sem):
        wid = lax.axis_index("s") * 2 + lax.axis_index("c")
        pltpu.sync_copy(src_hbm.at[pl.ds(wid * 2, 2)], idx_v)
        for j in range(2):
            pltpu.async_copy(x_hbm.at[idx_v.at[j]],
                             rows_v.at[pl.ds(j * 128, 128)], sem).wait()
        pltpu.sync_copy(rows_v, out_hbm.at[pl.ds(wid * _EPW, _EPW)])

    return g


def _make_scatter():
    @functools.partial(
        pl.kernel, mesh=_sc_mesh(),
        out_type=jax.ShapeDtypeStruct((_N, _W), jnp.float32),
        scratch_types=[
            pltpu.VMEM_SHARED((_NH + 8, _W), jnp.float32),
            pltpu.VMEM((_EPT, _W), jnp.float32),
            pltpu.VMEM((4, 128), jnp.int32),
            pltpu.SemaphoreType.DMA,
        ],
    )
    def s(msg_hbm, dst_hbm, root_hbm, out_hbm, acc, buf, idx_v, sem):
        c = lax.axis_index("c")
        sid = lax.axis_index("s")
        row0 = c * _NH + sid * _NPT
        pltpu.sync_copy(root_hbm.at[pl.ds(row0, _NPT)], buf.at[pl.ds(0, _NPT)])
        pltpu.sync_copy(buf.at[pl.ds(0, _NPT)], acc.at[pl.ds(sid * _NPT, _NPT)])
        plsc.subcore_barrier()
        pltpu.sync_copy(dst_hbm.at[pl.ds(c * 64 + sid * 4, 4)], idx_v)
        pltpu.sync_copy(msg_hbm.at[pl.ds(sid * _EPT, _EPT)], buf)
        for j in range(4):
            pltpu.sync_copy(buf.at[pl.ds(j * 128, 128)], acc.at[idx_v.at[j]],
                            add=True)
        plsc.subcore_barrier()
        pltpu.sync_copy(acc.at[pl.ds(sid * _NPT, _NPT)], buf.at[pl.ds(0, _NPT)])
        pltpu.sync_copy(buf.at[pl.ds(0, _NPT)], out_hbm.at[pl.ds(row0, _NPT)])

    return s


def _msg_body(xs_ref, x_ref, ea_ref, w0_ref, b0_ref, w1_ref, b1_ref,
              w2r_ref, b2r_ref, root_ref, bias_ref, msg_ref, rt_ref,
              *, din, dout, first):
    x = x_ref[...]
    if not first:
        x = jax.nn.relu(x)[:, :din]
    rt_ref[...] = x @ root_ref[...] + bias_ref[...]

    def body(b, _):
        sl = pl.ds(b * _BLK, _BLK)
        xs = xs_ref[sl, :]
        if not first:
            xs = jax.nn.relu(xs)
        xs = xs[:, :din]
        ea = ea_ref[sl, :]
        h = jax.nn.relu(ea @ w0_ref[...] + b0_ref[...])
        h = jax.nn.relu(h @ w1_ref[...] + b1_ref[...])
        msg = xs @ b2r_ref[...]
        if din == 4:
            t = h @ w2r_ref[...]
            for i in range(din):
                msg = msg + xs[:, i:i + 1] * t[:, i * dout:(i + 1) * dout]
        else:
            for k0 in range(0, 64, _KC):
                hc = h[:, k0:k0 + _KC]
                hrep = jnp.broadcast_to(hc[:, :, None],
                                        (_BLK, _KC, din)).reshape(_BLK, _KC * din)
                xst = jnp.broadcast_to(xs[:, None, :],
                                       (_BLK, _KC, din)).reshape(_BLK, _KC * din)
                msg = msg + (hrep * xst) @ w2r_ref[pl.ds(k0 * din, _KC * din), :]
        if dout < _W:
            msg = jnp.concatenate(
                [msg, jnp.zeros((_BLK, _W - dout), jnp.float32)], axis=1)
        msg_ref[sl, :] = msg
        return 0

    jax.lax.fori_loop(0, _E // _BLK, body, 0)


def _nnconv(x_raw, xs_raw, dst_adj, ea, p, scatter, first):
    din, dout = p['root'].shape
    w2r = p['w2'] if din == 4 else p['w2'].reshape(64 * din, dout)
    root_p = jnp.pad(p['root'], ((0, 0), (0, _W - dout)))
    bias_p = jnp.pad(p['bias'], (0, _W - dout)).reshape(1, _W)
    f = pl.pallas_call(
        functools.partial(_msg_body, din=din, dout=dout, first=first),
        out_shape=(jax.ShapeDtypeStruct((_E, _W), jnp.float32),
                   jax.ShapeDtypeStruct((_N, _W), jnp.float32)),
    )
    msg, rt = f(xs_raw, x_raw, ea,
                p['w0'], p['b0'].reshape(1, 64),
                p['w1'], p['b1'].reshape(1, 64),
                w2r, p['b2'].reshape(din, dout),
                root_p, bias_p)
    return scatter(msg, dst_adj, rt)


def _head_body(x1_ref, x2_ref, iin_ref, iout_ref, l1w_ref, l1b_ref, l2w_ref,
               l2b_ref, l3w_ref, l3b_ref, ow_ref, ob_ref, out_ref):
    wl = (l1w_ref[...] @ l2w_ref[...]) @ l3w_ref[...]
    bl = (l1b_ref[...] @ l2w_ref[...] + l2b_ref[...]) @ l3w_ref[...] + l3b_ref[...]
    iota_gn = jax.lax.broadcasted_iota(jnp.int32, (_G, _N), 1)
    oh_in = (jnp.reshape(iin_ref[...], (_G, 1)) == iota_gn).astype(jnp.float32)
    oh_out = (jnp.reshape(iout_ref[...], (_G, 1)) == iota_gn).astype(jnp.float32)
    x1 = jax.nn.relu(x1_ref[...])[:, :64]
    x2 = jax.nn.relu(x2_ref[...])[:, :64]
    a_in = jnp.concatenate([oh_in @ x1, oh_in @ x2], axis=1) @ wl + bl
    a_out = jnp.concatenate([oh_out @ x1, oh_out @ x2], axis=1) @ wl + bl
    cat = jnp.concatenate([a_in, a_out], axis=1)
    out_ref[...] = jnp.sum(cat * ow_ref[...], axis=1, keepdims=True) + ob_ref[...]


def kernel(x, edge_index, edge_attr1, edge_attr2, batch, params):
    src2d = edge_index[0].reshape(_E // 128, 128)
    dst = edge_index[1]
    dst_adj = jnp.concatenate([
        jnp.where((dst >= c * _NH) & (dst < (c + 1) * _NH), dst - c * _NH, _NH)
        for c in (0, 1)
    ]).astype(jnp.int32).reshape(2 * (_E // 128), 128)
    counts = jnp.bincount(batch, length=_G)
    starts = (jnp.cumsum(counts) - counts).astype(jnp.int32)

    x0p = jnp.pad(x, ((0, 0), (0, _W - x.shape[1])))
    gather = _make_gather()
    scatter = _make_scatter()

    def chain(ea, names):
        xc = x
        xc_pad = x0p
        first = True
        for name in names:
            xs = gather(xc_pad, src2d)
            xc = _nnconv(xc, xs, dst_adj, ea, params[name], scatter, first)
            xc_pad = xc
            first = False
        return xc

    x1 = chain(edge_attr1, ('c1a', 'c1b', 'c1c', 'c1d'))
    x2 = chain(edge_attr2, ('c2a', 'c2b', 'c2c', 'c2d'))

    head = pl.pallas_call(
        _head_body,
        out_shape=jax.ShapeDtypeStruct((_G, 1), jnp.float32),
    )
    return head(x1, x2, starts, starts + 1,
                params['lin1_w'], params['lin1_b'].reshape(1, 128),
                params['lin2_w'], params['lin2_b'].reshape(1, 64),
                params['lin3_w'], params['lin3_b'].reshape(1, 64),
                params['out_w'].reshape(1, 128), params['out_b'].reshape(1, 1))


# speedup vs baseline: 2.0745x; 1.0017x over previous
import functools

import jax
import jax.numpy as jnp
from jax import lax
from jax.experimental import pallas as pl
from jax.experimental.pallas import tpu as pltpu
from jax.experimental.pallas import tpu_sc as plsc

_N = 2048
_E = 8192
_G = 64
_W = 128
_BLK = 512
_KC = 16

_NW = 32
_EPW = _E // _NW
_EPT = _E // 16
_NH = _N // 2
_NPT = _NH // 16


def _sc_mesh():
    return plsc.VectorSubcoreMesh(core_axis_name="c", subcore_axis_name="s")


def _make_gather():
    @functools.partial(
        pl.kernel, mesh=_sc_mesh(),
        out_type=jax.ShapeDtypeStruct((_E, _W), jnp.float32),
        scratch_types=[
            pltpu.VMEM((2, 128), jnp.int32),
            pltpu.VMEM((_EPW, _W), jnp.float32),
            pltpu.SemaphoreType.DMA,
        ],
    )
    def g(x_hbm, src_hbm, out_hbm, idx_v, rows_v, sem):
        wid = lax.axis_index("s") * 2 + lax.axis_index("c")
        pltpu.sync_copy(src_hbm.at[pl.ds(wid * 2, 2)], idx_v)
        for j in range(2):
            pltpu.async_copy(x_hbm.at[idx_v.at[j]],
                             rows_v.at[pl.ds(j * 128, 128)], sem).wait()
        pltpu.sync_copy(rows_v, out_hbm.at[pl.ds(wid * _EPW, _EPW)])

    return g


def _make_scatter():
    @functools.partial(
        pl.kernel, mesh=_sc_mesh(),
        out_type=jax.ShapeDtypeStruct((_N, _W), jnp.float32),
        scratch_types=[
            pltpu.VMEM_SHARED((_NH + 8, _W), jnp.float32),
            pltpu.VMEM((_EPT, _W), jnp.float32),
            pltpu.VMEM((4, 128), jnp.int32),
            pltpu.SemaphoreType.DMA,
        ],
    )
    def s(msg_hbm, dst_hbm, root_hbm, out_hbm, acc, buf, idx_v, sem):
        c = lax.axis_index("c")
        sid = lax.axis_index("s")
        row0 = c * _NH + sid * _NPT
        pltpu.sync_copy(root_hbm.at[pl.ds(row0, _NPT)], buf.at[pl.ds(0, _NPT)])
        pltpu.sync_copy(buf.at[pl.ds(0, _NPT)], acc.at[pl.ds(sid * _NPT, _NPT)])
        plsc.subcore_barrier()
        pltpu.sync_copy(dst_hbm.at[pl.ds(c * 64 + sid * 4, 4)], idx_v)
        pltpu.sync_copy(msg_hbm.at[pl.ds(sid * _EPT, _EPT)], buf)
        for j in range(4):
            pltpu.sync_copy(buf.at[pl.ds(j * 128, 128)], acc.at[idx_v.at[j]],
                            add=True)
        plsc.subcore_barrier()
        pltpu.sync_copy(acc.at[pl.ds(sid * _NPT, _NPT)], buf.at[pl.ds(0, _NPT)])
        pltpu.sync_copy(buf.at[pl.ds(0, _NPT)], out_hbm.at[pl.ds(row0, _NPT)])

    return s


def _msg_body(xs_ref, x_ref, ea_ref, w0_ref, b0_ref, w1_ref, b1_ref,
              w2r_ref, b2r_ref, root_ref, bias_ref, msg_ref, rt_ref,
              *, din, dout, first):
    x = x_ref[...]
    if not first:
        x = jax.nn.relu(x)[:, :din]
    rt_ref[...] = x @ root_ref[...] + bias_ref[...]

    def body(b, _):
        sl = pl.ds(b * _BLK, _BLK)
        xs = xs_ref[sl, :]
        if not first:
            xs = jax.nn.relu(xs)
        xs = xs[:, :din]
        ea = ea_ref[sl, :]
        h = jax.nn.relu(ea @ w0_ref[...] + b0_ref[...])
        h = jax.nn.relu(h @ w1_ref[...] + b1_ref[...])
        msg = xs @ b2r_ref[...]
        if din == 4:
            t = h @ w2r_ref[...]
            for i in range(din):
                msg = msg + xs[:, i:i + 1] * t[:, i * dout:(i + 1) * dout]
        else:
            for k0 in range(0, 64, _KC):
                hc = h[:, k0:k0 + _KC]
                hrep = jnp.broadcast_to(hc[:, :, None],
                                        (_BLK, _KC, din)).reshape(_BLK, _KC * din)
                xst = jnp.broadcast_to(xs[:, None, :],
                                       (_BLK, _KC, din)).reshape(_BLK, _KC * din)
                msg = msg + (hrep * xst) @ w2r_ref[pl.ds(k0 * din, _KC * din), :]
        if dout < _W:
            msg = jnp.concatenate(
                [msg, jnp.zeros((_BLK, _W - dout), jnp.float32)], axis=1)
        msg_ref[sl, :] = msg
        return 0

    jax.lax.fori_loop(0, _E // _BLK, body, 0)


def _nnconv(x_raw, xs_raw, dst_adj, ea, p, scatter, first):
    din, dout = p['root'].shape
    w2r = p['w2'] if din == 4 else p['w2'].reshape(64 * din, dout)
    root_p = jnp.pad(p['root'], ((0, 0), (0, _W - dout)))
    bias_p = jnp.pad(p['bias'], (0, _W - dout)).reshape(1, _W)
    f = pl.pallas_call(
        functools.partial(_msg_body, din=din, dout=dout, first=first),
        out_shape=(jax.ShapeDtypeStruct((_E, _W), jnp.float32),
                   jax.ShapeDtypeStruct((_N, _W), jnp.float32)),
    )
    msg, rt = f(xs_raw, x_raw, ea,
                p['w0'], p['b0'].reshape(1, 64),
                p['w1'], p['b1'].reshape(1, 64),
                w2r, p['b2'].reshape(din, dout),
                root_p, bias_p)
    return scatter(msg, dst_adj, rt)


def _head_body(x1_ref, x2_ref, iin_ref, iout_ref, l1w_ref, l1b_ref, l2w_ref,
               l2b_ref, l3w_ref, l3b_ref, ow_ref, ob_ref, out_ref):
    wl = (l1w_ref[...] @ l2w_ref[...]) @ l3w_ref[...]
    bl = (l1b_ref[...] @ l2w_ref[...] + l2b_ref[...]) @ l3w_ref[...] + l3b_ref[...]
    iota_gn = jax.lax.broadcasted_iota(jnp.int32, (_G, _N), 1)
    oh_in = (jnp.reshape(iin_ref[...], (_G, 1)) == iota_gn).astype(jnp.float32)
    oh_out = (jnp.reshape(iout_ref[...], (_G, 1)) == iota_gn).astype(jnp.float32)
    x1 = jax.nn.relu(x1_ref[...])[:, :64]
    x2 = jax.nn.relu(x2_ref[...])[:, :64]
    a_in = jnp.concatenate([oh_in @ x1, oh_in @ x2], axis=1) @ wl + bl
    a_out = jnp.concatenate([oh_out @ x1, oh_out @ x2], axis=1) @ wl + bl
    cat = jnp.concatenate([a_in, a_out], axis=1)
    out_ref[...] = jnp.sum(cat * ow_ref[...], axis=1, keepdims=True) + ob_ref[...]


def kernel(x, edge_index, edge_attr1, edge_attr2, batch, params):
    src2d = edge_index[0].reshape(_E // 128, 128)
    dst = edge_index[1]
    dst_adj = jnp.concatenate([
        jnp.where((dst >= c * _NH) & (dst < (c + 1) * _NH), dst - c * _NH, _NH)
        for c in (0, 1)
    ]).astype(jnp.int32).reshape(2 * (_E // 128), 128)
    counts = jnp.bincount(batch, length=_G)
    starts = (jnp.cumsum(counts) - counts).astype(jnp.int32)

    x0p = jnp.pad(x, ((0, 0), (0, _W - x.shape[1])))
    gather = _make_gather()
    scatter = _make_scatter()

    names1 = ('c1a', 'c1b', 'c1c', 'c1d')
    names2 = ('c2a', 'c2b', 'c2c', 'c2d')
    x1 = x2 = x
    x1_pad = x2_pad = x0p
    first = True
    for n1, n2 in zip(names1, names2):
        xs1 = gather(x1_pad, src2d)
        xs2 = gather(x2_pad, src2d)
        x1 = _nnconv(x1, xs1, dst_adj, edge_attr1, params[n1], scatter, first)
        x2 = _nnconv(x2, xs2, dst_adj, edge_attr2, params[n2], scatter, first)
        x1_pad, x2_pad = x1, x2
        first = False

    head = pl.pallas_call(
        _head_body,
        out_shape=jax.ShapeDtypeStruct((_G, 1), jnp.float32),
    )
    return head(x1, x2, starts, starts + 1,
                params['lin1_w'], params['lin1_b'].reshape(1, 128),
                params['lin2_w'], params['lin2_b'].reshape(1, 64),
                params['lin3_w'], params['lin3_b'].reshape(1, 64),
                params['out_w'].reshape(1, 128), params['out_b'].reshape(1, 1))


# speedup vs baseline: 2.1961x; 1.0586x over previous
import functools

import jax
import jax.numpy as jnp
from jax import lax
from jax.experimental import pallas as pl
from jax.experimental.pallas import tpu as pltpu
from jax.experimental.pallas import tpu_sc as plsc

_N = 2048
_E = 8192
_G = 64
_W = 128
_BLK = 512
_KC = 16

_NW = 32
_EPW = _E // _NW
_EPT = _E // 16
_NH = _N // 2
_NPT = _NH // 16


def _sc_mesh():
    return plsc.VectorSubcoreMesh(core_axis_name="c", subcore_axis_name="s")


def _make_gather():
    @functools.partial(
        pl.kernel, mesh=_sc_mesh(),
        out_type=jax.ShapeDtypeStruct((_E, _W), jnp.float32),
        scratch_types=[
            pltpu.VMEM((2, 128), jnp.int32),
            pltpu.VMEM((_EPW, _W), jnp.float32),
            pltpu.SemaphoreType.DMA,
        ],
    )
    def g(x_hbm, src_hbm, out_hbm, idx_v, rows_v, sem):
        wid = lax.axis_index("s") * 2 + lax.axis_index("c")
        pltpu.sync_copy(src_hbm.at[pl.ds(wid * 2, 2)], idx_v)
        for j in range(2):
            pltpu.async_copy(x_hbm.at[idx_v.at[j]],
                             rows_v.at[pl.ds(j * 128, 128)], sem).wait()
        pltpu.sync_copy(rows_v, out_hbm.at[pl.ds(wid * _EPW, _EPW)])

    return g


def _make_scatter(with_gather):
    @functools.partial(
        pl.kernel, mesh=_sc_mesh(),
        out_type=(jax.ShapeDtypeStruct((_N, _W), jnp.float32),
                  jax.ShapeDtypeStruct((_E, _W), jnp.float32))
        if with_gather else jax.ShapeDtypeStruct((_N, _W), jnp.float32),
        scratch_types=[
            pltpu.VMEM_SHARED((_N, _W), jnp.float32),
            pltpu.VMEM((_EPT, _W), jnp.float32),
            pltpu.VMEM((4, 128), jnp.int32),
            pltpu.SemaphoreType.DMA,
        ],
    )
    def s(*refs):
        if with_gather:
            msg_hbm, dst_hbm, root_hbm, src_hbm, out_hbm, xs_hbm, acc, buf, idx_v, sem = refs
        else:
            msg_hbm, dst_hbm, root_hbm, out_hbm, acc, buf, idx_v, sem = refs
        c = lax.axis_index("c")
        sid = lax.axis_index("s")
        wid = sid * 2 + c
        npc = _N // 16
        pltpu.sync_copy(root_hbm.at[pl.ds(sid * npc, npc)],
                        buf.at[pl.ds(0, npc)])
        pltpu.sync_copy(buf.at[pl.ds(0, npc)], acc.at[pl.ds(sid * npc, npc)])
        plsc.subcore_barrier()
        pltpu.sync_copy(dst_hbm.at[pl.ds(sid * 4, 4)], idx_v)
        pltpu.sync_copy(msg_hbm.at[pl.ds(sid * _EPT, _EPT)], buf)
        for j in range(4):
            pltpu.sync_copy(buf.at[pl.ds(j * 128, 128)], acc.at[idx_v.at[j]],
                            add=True)
        plsc.subcore_barrier()
        nw = _N // 32
        pltpu.sync_copy(acc.at[pl.ds(wid * nw, nw)], buf.at[pl.ds(0, nw)])
        pltpu.sync_copy(buf.at[pl.ds(0, nw)], out_hbm.at[pl.ds(wid * nw, nw)])
        if with_gather:
            pltpu.sync_copy(src_hbm.at[pl.ds(wid * 2, 2)],
                            idx_v.at[pl.ds(0, 2)])
            for j in range(2):
                pltpu.async_copy(acc.at[idx_v.at[j]],
                                 buf.at[pl.ds(j * 128, 128)], sem).wait()
            pltpu.sync_copy(buf.at[pl.ds(0, _EPW)],
                            xs_hbm.at[pl.ds(wid * _EPW, _EPW)])

    return s


def _msg_body(xs_ref, x_ref, ea_ref, w0_ref, b0_ref, w1_ref, b1_ref,
              w2r_ref, b2r_ref, root_ref, bias_ref, msg_ref, rt_ref,
              *, din, dout, first):
    x = x_ref[...]
    if not first:
        x = jax.nn.relu(x)[:, :din]
    rt_ref[...] = x @ root_ref[...] + bias_ref[...]

    def body(b, _):
        sl = pl.ds(b * _BLK, _BLK)
        xs = xs_ref[sl, :]
        if not first:
            xs = jax.nn.relu(xs)
        xs = xs[:, :din]
        ea = ea_ref[sl, :]
        h = jax.nn.relu(ea @ w0_ref[...] + b0_ref[...])
        h = jax.nn.relu(h @ w1_ref[...] + b1_ref[...])
        msg = xs @ b2r_ref[...]
        if din == 4:
            t = h @ w2r_ref[...]
            for i in range(din):
                msg = msg + xs[:, i:i + 1] * t[:, i * dout:(i + 1) * dout]
        else:
            for k0 in range(0, 64, _KC):
                hc = h[:, k0:k0 + _KC]
                hrep = jnp.broadcast_to(hc[:, :, None],
                                        (_BLK, _KC, din)).reshape(_BLK, _KC * din)
                xst = jnp.broadcast_to(xs[:, None, :],
                                       (_BLK, _KC, din)).reshape(_BLK, _KC * din)
                msg = msg + (hrep * xst) @ w2r_ref[pl.ds(k0 * din, _KC * din), :]
        if dout < _W:
            msg = jnp.concatenate(
                [msg, jnp.zeros((_BLK, _W - dout), jnp.float32)], axis=1)
        msg_ref[sl, :] = msg
        return 0

    jax.lax.fori_loop(0, _E // _BLK, body, 0)


def _edge_msg(x_raw, xs_raw, ea, p, first):
    din, dout = p['root'].shape
    w2r = p['w2'] if din == 4 else p['w2'].reshape(64 * din, dout)
    root_p = jnp.pad(p['root'], ((0, 0), (0, _W - dout)))
    bias_p = jnp.pad(p['bias'], (0, _W - dout)).reshape(1, _W)
    f = pl.pallas_call(
        functools.partial(_msg_body, din=din, dout=dout, first=first),
        out_shape=(jax.ShapeDtypeStruct((_E, _W), jnp.float32),
                   jax.ShapeDtypeStruct((_N, _W), jnp.float32)),
    )
    return f(xs_raw, x_raw, ea,
             p['w0'], p['b0'].reshape(1, 64),
             p['w1'], p['b1'].reshape(1, 64),
             w2r, p['b2'].reshape(din, dout),
             root_p, bias_p)


def _head_body(x1_ref, x2_ref, iin_ref, iout_ref, l1w_ref, l1b_ref, l2w_ref,
               l2b_ref, l3w_ref, l3b_ref, ow_ref, ob_ref, out_ref):
    wl = (l1w_ref[...] @ l2w_ref[...]) @ l3w_ref[...]
    bl = (l1b_ref[...] @ l2w_ref[...] + l2b_ref[...]) @ l3w_ref[...] + l3b_ref[...]
    iota_gn = jax.lax.broadcasted_iota(jnp.int32, (_G, _N), 1)
    oh_in = (jnp.reshape(iin_ref[...], (_G, 1)) == iota_gn).astype(jnp.float32)
    oh_out = (jnp.reshape(iout_ref[...], (_G, 1)) == iota_gn).astype(jnp.float32)
    x1 = jax.nn.relu(x1_ref[...])[:, :64]
    x2 = jax.nn.relu(x2_ref[...])[:, :64]
    a_in = jnp.concatenate([oh_in @ x1, oh_in @ x2], axis=1) @ wl + bl
    a_out = jnp.concatenate([oh_out @ x1, oh_out @ x2], axis=1) @ wl + bl
    cat = jnp.concatenate([a_in, a_out], axis=1)
    out_ref[...] = jnp.sum(cat * ow_ref[...], axis=1, keepdims=True) + ob_ref[...]


def kernel(x, edge_index, edge_attr1, edge_attr2, batch, params):
    src2d = edge_index[0].reshape(_E // 128, 128)
    dst2d = edge_index[1].reshape(_E // 128, 128)
    counts = jnp.bincount(batch, length=_G)
    starts = (jnp.cumsum(counts) - counts).astype(jnp.int32)

    x0p = jnp.pad(x, ((0, 0), (0, _W - x.shape[1])))
    gather = _make_gather()
    scatter_g = _make_scatter(True)
    scatter_o = _make_scatter(False)

    names1 = ('c1a', 'c1b', 'c1c', 'c1d')
    names2 = ('c2a', 'c2b', 'c2c', 'c2d')
    x1 = x2 = x
    xs1 = gather(x0p, src2d)
    xs2 = gather(x0p, src2d)
    first = True
    for i, (n1, n2) in enumerate(zip(names1, names2)):
        msg1, rt1 = _edge_msg(x1, xs1, edge_attr1, params[n1], first)
        msg2, rt2 = _edge_msg(x2, xs2, edge_attr2, params[n2], first)
        if i < 3:
            x1, xs1 = scatter_g(msg1, dst2d, rt1, src2d)
            x2, xs2 = scatter_g(msg2, dst2d, rt2, src2d)
        else:
            x1 = scatter_o(msg1, dst2d, rt1)
            x2 = scatter_o(msg2, dst2d, rt2)
        first = False

    head = pl.pallas_call(
        _head_body,
        out_shape=jax.ShapeDtypeStruct((_G, 1), jnp.float32),
    )
    return head(x1, x2, starts, starts + 1,
                params['lin1_w'], params['lin1_b'].reshape(1, 128),
                params['lin2_w'], params['lin2_b'].reshape(1, 64),
                params['lin3_w'], params['lin3_b'].reshape(1, 64),
                params['out_w'].reshape(1, 128), params['out_b'].reshape(1, 1))


# speedup vs baseline: 2.2136x; 1.0080x over previous
import functools

import jax
import jax.numpy as jnp
from jax import lax
from jax.experimental import pallas as pl
from jax.experimental.pallas import tpu as pltpu
from jax.experimental.pallas import tpu_sc as plsc

_N = 2048
_E = 8192
_G = 64
_W = 128
_BLK = 1024
_KC = 32

_NW = 32
_EPW = _E // _NW
_EPT = _E // 16
_NH = _N // 2
_NPT = _NH // 16


def _sc_mesh():
    return plsc.VectorSubcoreMesh(core_axis_name="c", subcore_axis_name="s")


def _make_gather():
    @functools.partial(
        pl.kernel, mesh=_sc_mesh(),
        out_type=jax.ShapeDtypeStruct((_E, _W), jnp.float32),
        scratch_types=[
            pltpu.VMEM((2, 128), jnp.int32),
            pltpu.VMEM((_EPW, _W), jnp.float32),
            pltpu.SemaphoreType.DMA,
        ],
    )
    def g(x_hbm, src_hbm, out_hbm, idx_v, rows_v, sem):
        wid = lax.axis_index("s") * 2 + lax.axis_index("c")
        pltpu.sync_copy(src_hbm.at[pl.ds(wid * 2, 2)], idx_v)
        for j in range(2):
            pltpu.async_copy(x_hbm.at[idx_v.at[j]],
                             rows_v.at[pl.ds(j * 128, 128)], sem).wait()
        pltpu.sync_copy(rows_v, out_hbm.at[pl.ds(wid * _EPW, _EPW)])

    return g


def _make_scatter(with_gather):
    @functools.partial(
        pl.kernel, mesh=_sc_mesh(),
        out_type=(jax.ShapeDtypeStruct((_N, _W), jnp.float32),
                  jax.ShapeDtypeStruct((_E, _W), jnp.float32))
        if with_gather else jax.ShapeDtypeStruct((_N, _W), jnp.float32),
        scratch_types=[
            pltpu.VMEM_SHARED((_N, _W), jnp.float32),
            pltpu.VMEM((_EPT, _W), jnp.float32),
            pltpu.VMEM((4, 128), jnp.int32),
            pltpu.SemaphoreType.DMA,
        ],
    )
    def s(*refs):
        if with_gather:
            msg_hbm, dst_hbm, root_hbm, src_hbm, out_hbm, xs_hbm, acc, buf, idx_v, sem = refs
        else:
            msg_hbm, dst_hbm, root_hbm, out_hbm, acc, buf, idx_v, sem = refs
        c = lax.axis_index("c")
        sid = lax.axis_index("s")
        wid = sid * 2 + c
        npc = _N // 16
        pltpu.sync_copy(root_hbm.at[pl.ds(sid * npc, npc)],
                        buf.at[pl.ds(0, npc)])
        pltpu.sync_copy(buf.at[pl.ds(0, npc)], acc.at[pl.ds(sid * npc, npc)])
        plsc.subcore_barrier()
        pltpu.sync_copy(dst_hbm.at[pl.ds(sid * 4, 4)], idx_v)
        pltpu.sync_copy(msg_hbm.at[pl.ds(sid * _EPT, _EPT)], buf)
        for j in range(4):
            pltpu.sync_copy(buf.at[pl.ds(j * 128, 128)], acc.at[idx_v.at[j]],
                            add=True)
        plsc.subcore_barrier()
        nw = _N // 32
        pltpu.sync_copy(acc.at[pl.ds(wid * nw, nw)], buf.at[pl.ds(0, nw)])
        pltpu.sync_copy(buf.at[pl.ds(0, nw)], out_hbm.at[pl.ds(wid * nw, nw)])
        if with_gather:
            pltpu.sync_copy(src_hbm.at[pl.ds(wid * 2, 2)],
                            idx_v.at[pl.ds(0, 2)])
            for j in range(2):
                pltpu.async_copy(acc.at[idx_v.at[j]],
                                 buf.at[pl.ds(j * 128, 128)], sem).wait()
            pltpu.sync_copy(buf.at[pl.ds(0, _EPW)],
                            xs_hbm.at[pl.ds(wid * _EPW, _EPW)])

    return s


def _msg_body(xs_ref, x_ref, ea_ref, w0_ref, b0_ref, w1_ref, b1_ref,
              w2r_ref, b2r_ref, root_ref, bias_ref, msg_ref, rt_ref,
              *, din, dout, first):
    x = x_ref[...]
    if not first:
        x = jax.nn.relu(x)[:, :din]
    rt_ref[...] = x @ root_ref[...] + bias_ref[...]

    def body(b, _):
        sl = pl.ds(b * _BLK, _BLK)
        xs = xs_ref[sl, :]
        if not first:
            xs = jax.nn.relu(xs)
        xs = xs[:, :din]
        ea = ea_ref[sl, :]
        h = jax.nn.relu(ea @ w0_ref[...] + b0_ref[...])
        h = jax.nn.relu(h @ w1_ref[...] + b1_ref[...])
        msg = xs @ b2r_ref[...]
        if din == 4:
            t = h @ w2r_ref[...]
            for i in range(din):
                msg = msg + xs[:, i:i + 1] * t[:, i * dout:(i + 1) * dout]
        else:
            for k0 in range(0, 64, _KC):
                hc = h[:, k0:k0 + _KC]
                hrep = jnp.broadcast_to(hc[:, :, None],
                                        (_BLK, _KC, din)).reshape(_BLK, _KC * din)
                xst = jnp.broadcast_to(xs[:, None, :],
                                       (_BLK, _KC, din)).reshape(_BLK, _KC * din)
                msg = msg + (hrep * xst) @ w2r_ref[pl.ds(k0 * din, _KC * din), :]
        if dout < _W:
            msg = jnp.concatenate(
                [msg, jnp.zeros((_BLK, _W - dout), jnp.float32)], axis=1)
        msg_ref[sl, :] = msg
        return 0

    jax.lax.fori_loop(0, _E // _BLK, body, 0)


def _edge_msg(x_raw, xs_raw, ea, p, first):
    din, dout = p['root'].shape
    w2r = p['w2'] if din == 4 else p['w2'].reshape(64 * din, dout)
    root_p = jnp.pad(p['root'], ((0, 0), (0, _W - dout)))
    bias_p = jnp.pad(p['bias'], (0, _W - dout)).reshape(1, _W)
    f = pl.pallas_call(
        functools.partial(_msg_body, din=din, dout=dout, first=first),
        out_shape=(jax.ShapeDtypeStruct((_E, _W), jnp.float32),
                   jax.ShapeDtypeStruct((_N, _W), jnp.float32)),
    )
    return f(xs_raw, x_raw, ea,
             p['w0'], p['b0'].reshape(1, 64),
             p['w1'], p['b1'].reshape(1, 64),
             w2r, p['b2'].reshape(din, dout),
             root_p, bias_p)


def _head_body(x1_ref, x2_ref, iin_ref, iout_ref, l1w_ref, l1b_ref, l2w_ref,
               l2b_ref, l3w_ref, l3b_ref, ow_ref, ob_ref, out_ref):
    wl = (l1w_ref[...] @ l2w_ref[...]) @ l3w_ref[...]
    bl = (l1b_ref[...] @ l2w_ref[...] + l2b_ref[...]) @ l3w_ref[...] + l3b_ref[...]
    iota_gn = jax.lax.broadcasted_iota(jnp.int32, (_G, _N), 1)
    oh_in = (jnp.reshape(iin_ref[...], (_G, 1)) == iota_gn).astype(jnp.float32)
    oh_out = (jnp.reshape(iout_ref[...], (_G, 1)) == iota_gn).astype(jnp.float32)
    x1 = jax.nn.relu(x1_ref[...])[:, :64]
    x2 = jax.nn.relu(x2_ref[...])[:, :64]
    a_in = jnp.concatenate([oh_in @ x1, oh_in @ x2], axis=1) @ wl + bl
    a_out = jnp.concatenate([oh_out @ x1, oh_out @ x2], axis=1) @ wl + bl
    cat = jnp.concatenate([a_in, a_out], axis=1)
    out_ref[...] = jnp.sum(cat * ow_ref[...], axis=1, keepdims=True) + ob_ref[...]


def kernel(x, edge_index, edge_attr1, edge_attr2, batch, params):
    src2d = edge_index[0].reshape(_E // 128, 128)
    dst2d = edge_index[1].reshape(_E // 128, 128)
    counts = jnp.bincount(batch, length=_G)
    starts = (jnp.cumsum(counts) - counts).astype(jnp.int32)

    x0p = jnp.pad(x, ((0, 0), (0, _W - x.shape[1])))
    gather = _make_gather()
    scatter_g = _make_scatter(True)
    scatter_o = _make_scatter(False)

    names1 = ('c1a', 'c1b', 'c1c', 'c1d')
    names2 = ('c2a', 'c2b', 'c2c', 'c2d')
    x1 = x2 = x
    xs1 = gather(x0p, src2d)
    xs2 = gather(x0p, src2d)
    first = True
    for i, (n1, n2) in enumerate(zip(names1, names2)):
        msg1, rt1 = _edge_msg(x1, xs1, edge_attr1, params[n1], first)
        msg2, rt2 = _edge_msg(x2, xs2, edge_attr2, params[n2], first)
        if i < 3:
            x1, xs1 = scatter_g(msg1, dst2d, rt1, src2d)
            x2, xs2 = scatter_g(msg2, dst2d, rt2, src2d)
        else:
            x1 = scatter_o(msg1, dst2d, rt1)
            x2 = scatter_o(msg2, dst2d, rt2)
        first = False

    head = pl.pallas_call(
        _head_body,
        out_shape=jax.ShapeDtypeStruct((_G, 1), jnp.float32),
    )
    return head(x1, x2, starts, starts + 1,
                params['lin1_w'], params['lin1_b'].reshape(1, 128),
                params['lin2_w'], params['lin2_b'].reshape(1, 64),
                params['lin3_w'], params['lin3_b'].reshape(1, 64),
                params['out_w'].reshape(1, 128), params['out_b'].reshape(1, 1))
